# Initial kernel scaffold; baseline (speedup 1.0000x reference)
import jax
import jax.numpy as jnp
from jax.experimental import pallas as pl


def kernel(inputs, encode_weight, decode_weight, decode_bias):
    raise NotImplementedError("write your pallas kernel here")


# speedup vs baseline: 8.0331x; 8.0331x over previous
import functools

import jax
import jax.numpy as jnp
from jax import lax
from jax.experimental import pallas as pl
from jax.experimental.pallas import tpu as pltpu
from jax.experimental.pallas import tpu_sc as plsc

N = 34
D = 128
W = 8
B = 16384
NP = N * N

NC = 2
NS = 16
NW = NC * NS
L = 16
C = B // NW
G = C // L


def _pair_table_body(e_ref, d_ref, b_ref, out_ref):
    m = lax.dot_general(e_ref[...], d_ref[...], (((1,), (1,)), ((), ())),
                        preferred_element_type=jnp.float32)
    m = (m + b_ref[...]) * 0.125
    out_ref[...] = m[:, None, :] + m[None, :, :]


def _sc_body(t2_hbm, idx_hbm, out_hbm, tab_v, idx_v, res_v, sem):
    wid = lax.axis_index("s") * NC + lax.axis_index("c")
    base = wid * C

    cp_tab = pltpu.async_copy(t2_hbm, tab_v, sem)
    cp_idx = [pltpu.async_copy(idx_hbm.at[w, pl.ds(base, C)], idx_v.at[w], sem)
              for w in range(W)]
    cp_tab.wait()
    for cp in cp_idx:
        cp.wait()

    lane = lax.broadcasted_iota(jnp.int32, (L,), 0)

    def group(g, carry):
        b0 = g * L
        iv = [idx_v[w, pl.ds(b0, L)] for w in range(W)]
        pv = [iv[2 * i] * N + iv[2 * i + 1] for i in range(W // 2)]
        sbase = (b0 + lane) * N
        fv = [p * N for p in pv]
        for c in range(N):
            g0 = plsc.load_gather(tab_v, [fv[0] + c])
            g1 = plsc.load_gather(tab_v, [fv[1] + c])
            g2 = plsc.load_gather(tab_v, [fv[2] + c])
            g3 = plsc.load_gather(tab_v, [fv[3] + c])
            plsc.store_scatter(res_v, [sbase + c], (g0 + g1) + (g2 + g3))
        return carry

    lax.fori_loop(0, G, group, 0)

    pltpu.sync_copy(res_v, out_hbm.at[pl.ds(base * N, C * N)])


def kernel(inputs, encode_weight, decode_weight, decode_bias):
    t2 = pl.pallas_call(
        _pair_table_body,
        out_shape=jax.ShapeDtypeStruct((N, N, N), jnp.float32),
    )(encode_weight, decode_weight, decode_bias.reshape(1, N))
    t2 = t2.reshape(NP, N)

    mesh = plsc.VectorSubcoreMesh(core_axis_name="c", subcore_axis_name="s",
                                  num_cores=NC, num_subcores=NS)
    sc = functools.partial(
        pl.kernel,
        out_type=jax.ShapeDtypeStruct((B * N,), jnp.float32),
        mesh=mesh,
        scratch_types=[
            pltpu.VMEM((NP * N,), jnp.float32),
            pltpu.VMEM((W, C), jnp.int32),
            pltpu.VMEM((C * N,), jnp.float32),
            pltpu.SemaphoreType.DMA,
        ],
        compiler_params=pltpu.CompilerParams(needs_layout_passes=False),
    )(_sc_body)
    flat = sc(t2.reshape(NP * N), inputs)
    return flat.reshape(B, N)
